# baseline (speedup 1.0000x reference)
import functools

import jax
import jax.numpy as jnp
from jax import lax
from jax.experimental import pallas as pl
from jax.experimental.pallas import tpu as pltpu
from jax.experimental.pallas import tpu_sc as plsc

N = 10000
E = 320000
D = 128
HID = 32
NC = 2
NS = 16
NW = NC * NS
NPAD = 10240
SL = NPAD // NS
EPAD = 327680
EW = EPAD // NW
CHUNK = 128
NCHUNK = EW // CHUNK

_HIGH = lax.Precision.HIGHEST


def _dense_pre_body(inp, wfc, w0, b0, deg, x_o, aft_o, newx_o, rdh_o, rdeg_o):
    a = inp[...]
    x = lax.dot_general(a, wfc[...], (((1,), (1,)), ((), ())), precision=_HIGH)
    af = lax.dot_general(a, w0[...], (((1,), (1,)), ((), ())), precision=_HIGH)
    af = af + b0[...]
    d = deg[...]
    dh = jnp.sqrt(d)
    x_o[...] = x
    aft_o[...] = af.T
    newx_o[...] = x / dh
    rdh_o[...] = 1.0 / dh
    rdeg_o[...] = 1.0 / d


def _dense_pre(inp, wfc, w0, b0, deg):
    blk = 1024
    grid = NPAD // blk
    return pl.pallas_call(
        _dense_pre_body,
        grid=(grid,),
        in_specs=[
            pl.BlockSpec((blk, D), lambda i: (i, 0)),
            pl.BlockSpec((D, D), lambda i: (0, 0)),
            pl.BlockSpec((HID, D), lambda i: (0, 0)),
            pl.BlockSpec((1, HID), lambda i: (0, 0)),
            pl.BlockSpec((blk, 1), lambda i: (i, 0)),
        ],
        out_specs=[
            pl.BlockSpec((blk, D), lambda i: (i, 0)),
            pl.BlockSpec((HID, blk), lambda i: (0, i)),
            pl.BlockSpec((blk, D), lambda i: (i, 0)),
            pl.BlockSpec((blk, 1), lambda i: (i, 0)),
            pl.BlockSpec((blk, 1), lambda i: (i, 0)),
        ],
        out_shape=[
            jax.ShapeDtypeStruct((N, D), jnp.float32),
            jax.ShapeDtypeStruct((HID, NPAD), jnp.float32),
            jax.ShapeDtypeStruct((N, D), jnp.float32),
            jax.ShapeDtypeStruct((N, 1), jnp.float32),
            jax.ShapeDtypeStruct((N, 1), jnp.float32),
        ],
    )(inp, wfc, w0, b0, deg)


def _edge_mlp_body(lt_r, rt_r, w1, b1, w2, b2, s_o):
    l = lt_r[...]
    r = rt_r[...]
    feat = jnp.concatenate([l, r, jnp.abs(l - r),
                            jnp.ones((1, l.shape[1]), jnp.float32)], axis=0)
    w1e = jnp.concatenate([w1[...], b1[...]], axis=1)
    h = lax.dot_general(w1e, feat, (((1,), (0,)), ((), ())), precision=_HIGH)
    h = jnp.maximum(h, 0.0)
    w2p = jnp.concatenate([w2[...], jnp.zeros((7, HID), jnp.float32)], axis=0)
    z = lax.dot_general(w2p, h, (((1,), (0,)), ((), ())), precision=_HIGH)
    z = z + b2[0, 0]
    s_o[...] = 1.0 / (1.0 + jnp.exp(-z[0:1, :]))


def _edge_mlp(lt, rt, w1, b1, w2, b2):
    blk = 8192
    grid = EPAD // blk
    return pl.pallas_call(
        _edge_mlp_body,
        grid=(grid,),
        in_specs=[
            pl.BlockSpec((HID, blk), lambda i: (0, i)),
            pl.BlockSpec((HID, blk), lambda i: (0, i)),
            pl.BlockSpec((HID, 3 * HID), lambda i: (0, 0)),
            pl.BlockSpec((HID, 1), lambda i: (0, 0)),
            pl.BlockSpec((1, HID), lambda i: (0, 0)),
            pl.BlockSpec((1, 1), lambda i: (0, 0)),
        ],
        out_specs=pl.BlockSpec((1, blk), lambda i: (0, i)),
        out_shape=jax.ShapeDtypeStruct((1, EPAD), jnp.float32),
    )(lt, rt, w1, b1, w2, b2)


def _final_aggr_body(p0, p1, x_r, pef, rdh, out):
    a = (p0[...] + p1[...]) * rdh[...]
    p = pef[...]
    out[...] = a * p + x_r[...] * (1.0 - p)


def _final_aggr(aggrp, x, pef, rdh):
    blk = 1024
    grid = NPAD // blk
    return pl.pallas_call(
        _final_aggr_body,
        grid=(grid,),
        in_specs=[
            pl.BlockSpec((blk, D), lambda i: (i, 0)),
            pl.BlockSpec((blk, D), lambda i: (i + NPAD // 1024, 0)),
            pl.BlockSpec((blk, D), lambda i: (i, 0)),
            pl.BlockSpec((blk, 1), lambda i: (i, 0)),
            pl.BlockSpec((blk, 1), lambda i: (i, 0)),
        ],
        out_specs=pl.BlockSpec((blk, D), lambda i: (i, 0)),
        out_shape=jax.ShapeDtypeStruct((N, D), jnp.float32),
    )(aggrp, aggrp, x, pef, rdh)


def _edge_out_body(e2p, rdeg, out):
    v = e2p[...]
    e2 = (v[0] + v[1])[None, :N]
    out[...] = rdeg[...] * e2


def _edge_out(e2p, rdeg):
    blk = 400
    grid = N // blk
    return pl.pallas_call(
        _edge_out_body,
        grid=(grid,),
        in_specs=[
            pl.BlockSpec((2, NPAD), lambda i: (0, 0)),
            pl.BlockSpec((blk, 1), lambda i: (i, 0)),
        ],
        out_specs=pl.BlockSpec((blk, N), lambda i: (i, 0)),
        out_shape=jax.ShapeDtypeStruct((N, N), jnp.float32),
    )(e2p, rdeg)


_Z16 = functools.partial(jnp.zeros, (16,), jnp.float32)


def _worker_base():
    c = lax.axis_index("c")
    s = lax.axis_index("s")
    wid = c * NS + s
    return c, s, wid * EW


def _zero_slice(zb, acc, s, width):
    n = zb.shape[0]
    if len(zb.shape) == 1:
        for j in range(n // 16):
            zb[pl.ds(j * 16, 16)] = _Z16()
        @pl.loop(0, width // n)
        def _(r):
            pltpu.sync_copy(zb, acc.at[pl.ds(s * width + r * n, n)])
    else:
        for i in range(n):
            for j in range(zb.shape[1] // 16):
                zb[i, pl.ds(j * 16, 16)] = _Z16()
        @pl.loop(0, width // n)
        def _(r):
            pltpu.sync_copy(zb, acc.at[pl.ds(s * width + r * n, n)])


def _edge_gather_body(rowp, colp, aft, lt_o, rt_o, af8,
                      i0, i1, ob0, ob1, si0, si1, so0, so1):
    c = lax.axis_index("c")
    s = lax.axis_index("s")
    wid = c * NS + s
    fg = wid % (HID // 8)
    er = wid // (HID // 8)
    erw = EPAD // 8
    echunk = 2048
    nch = erw // echunk
    frow = pl.multiple_of(fg * 8, 8)
    pltpu.sync_copy(aft.at[pl.ds(frow, 8), :], af8)

    idxb = (i0, i1)
    obs = (ob0, ob1)
    sis = (si0, si1)
    sos = (so0, so1)

    for idx_src, dst in ((rowp, lt_o), (colp, rt_o)):
        base = pl.multiple_of(er * erw, echunk)
        pltpu.sync_copy(idx_src.at[pl.ds(base, echunk)], i0)

        @pl.loop(0, nch // 2)
        def _(t):
            for b in range(2):
                k = t * 2 + b
                ib, ob_, si, so = idxb[b], obs[b], sis[b], sos[b]
                inx, sinx = idxb[b ^ 1], sis[b ^ 1]
                nxt = pl.multiple_of(base + (k + 1) * echunk, echunk)
                if b == 0:
                    pltpu.async_copy(idx_src.at[pl.ds(nxt, echunk)], inx, sinx)
                else:
                    @pl.when(t < nch // 2 - 1)
                    def _():
                        pltpu.async_copy(idx_src.at[pl.ds(nxt, echunk)],
                                         inx, sinx)
                if b == 1:
                    pltpu.make_async_copy(
                        idx_src.at[pl.ds(base, echunk)], ib, si).wait()
                else:
                    @pl.when(t > 0)
                    def _():
                        pltpu.make_async_copy(
                            idx_src.at[pl.ds(base, echunk)], ib, si).wait()
                @pl.when(k >= 2)
                def _():
                    pltpu.make_async_copy(
                        ob_, dst.at[pl.ds(frow, 8), pl.ds(base, echunk)],
                        so).wait()

                @pl.loop(0, echunk // 16)
                def _(g):
                    iv = ib[pl.ds(g * 16, 16)]
                    for kk in range(8):
                        kv = jnp.full((16,), kk, jnp.int32)
                        ob_[kk, pl.ds(g * 16, 16)] = plsc.load_gather(
                            af8, [kv, iv])

                off = pl.multiple_of(base + k * echunk, echunk)
                pltpu.async_copy(
                    ob_, dst.at[pl.ds(frow, 8), pl.ds(off, echunk)], so)

        pltpu.make_async_copy(
            ob0, dst.at[pl.ds(frow, 8), pl.ds(base, echunk)], so0).wait()
        pltpu.make_async_copy(
            ob1, dst.at[pl.ds(frow, 8), pl.ds(base, echunk)], so1).wait()


def _edge_gather(rowp, colp, aft):
    mesh = plsc.VectorSubcoreMesh(core_axis_name="c", subcore_axis_name="s")
    f = pl.kernel(
        _edge_gather_body,
        out_type=[
            jax.ShapeDtypeStruct((HID, EPAD), jnp.float32),
            jax.ShapeDtypeStruct((HID, EPAD), jnp.float32),
        ],
        mesh=mesh,
        compiler_params=pltpu.CompilerParams(needs_layout_passes=False),
        scratch_types=[
            pltpu.VMEM((8, NPAD), jnp.float32),
            pltpu.VMEM((2048,), jnp.int32),
            pltpu.VMEM((2048,), jnp.int32),
            pltpu.VMEM((8, 2048), jnp.float32),
            pltpu.VMEM((8, 2048), jnp.float32),
            pltpu.SemaphoreType.DMA,
            pltpu.SemaphoreType.DMA,
            pltpu.SemaphoreType.DMA,
            pltpu.SemaphoreType.DMA,
        ],
    )
    return f(rowp, colp, aft)


def _seg_e1_body(rowp2, sflat, e1p, acc, rowa, vala, zb, sem):
    c, s, base = _worker_base()
    cb = (c * NS + s) * NCHUNK
    pltpu.async_copy(rowp2.at[pl.ds(cb, NCHUNK), :], rowa, sem)
    pltpu.async_copy(sflat.at[pl.ds(base, EW)], vala, sem)
    _zero_slice(zb, acc, s, SL)
    pltpu.make_async_copy(rowp2.at[pl.ds(cb, NCHUNK), :], rowa, sem).wait()
    pltpu.make_async_copy(sflat.at[pl.ds(base, EW)], vala, sem).wait()
    plsc.subcore_barrier()

    @pl.loop(0, NCHUNK)
    def _(k):
        off = pl.multiple_of(k * CHUNK, CHUNK)
        pltpu.async_copy(vala.at[pl.ds(off, CHUNK)], acc.at[rowa.at[k]],
                         sem, add=True)

    pltpu.make_async_copy(sflat.at[pl.ds(base, EW)], vala, sem).wait()
    plsc.subcore_barrier()
    pltpu.sync_copy(acc.at[pl.ds(s * SL, SL)],
                    e1p.at[pl.ds(c * NPAD + s * SL, SL)])


def _seg_e1(rowp2, sflat):
    mesh = plsc.VectorSubcoreMesh(core_axis_name="c", subcore_axis_name="s")
    f = pl.kernel(
        _seg_e1_body,
        out_type=jax.ShapeDtypeStruct((NC * NPAD,), jnp.float32),
        mesh=mesh,
        scratch_types=[
            pltpu.VMEM_SHARED((NPAD,), jnp.float32),
            pltpu.VMEM((NCHUNK, CHUNK), jnp.int32),
            pltpu.VMEM((EW,), jnp.float32),
            pltpu.VMEM((SL // 4,), jnp.float32),
            pltpu.SemaphoreType.DMA,
        ],
    )
    return f(rowp2, sflat)


def _seg_e2_body(rowp2, colp, e1p, e2p, acc, e1a, e1b, cola, rowa, vala,
                 zb, sem):
    c, s, base = _worker_base()
    cb = (c * NS + s) * NCHUNK
    pltpu.async_copy(e1p.at[pl.ds(0, NPAD)], e1a, sem)
    pltpu.async_copy(e1p.at[pl.ds(NPAD, NPAD)], e1b, sem)
    pltpu.async_copy(colp.at[pl.ds(base, EW)], cola, sem)
    pltpu.async_copy(rowp2.at[pl.ds(cb, NCHUNK), :], rowa, sem)
    _zero_slice(zb, acc, s, SL)
    pltpu.make_async_copy(e1p.at[pl.ds(0, NPAD)], e1a, sem).wait()
    pltpu.make_async_copy(e1p.at[pl.ds(NPAD, NPAD)], e1b, sem).wait()
    pltpu.make_async_copy(colp.at[pl.ds(base, EW)], cola, sem).wait()
    pltpu.make_async_copy(rowp2.at[pl.ds(cb, NCHUNK), :], rowa, sem).wait()

    @pl.loop(0, EW // 16)
    def _(g):
        cv = cola[pl.ds(g * 16, 16)]
        vala[pl.ds(g * 16, 16)] = (plsc.load_gather(e1a, [cv]) +
                                   plsc.load_gather(e1b, [cv]))

    plsc.subcore_barrier()

    @pl.loop(0, NCHUNK)
    def _(k):
        off = pl.multiple_of(k * CHUNK, CHUNK)
        pltpu.async_copy(vala.at[pl.ds(off, CHUNK)], acc.at[rowa.at[k]],
                         sem, add=True)

    pltpu.make_async_copy(colp.at[pl.ds(base, EW)], vala, sem).wait()
    plsc.subcore_barrier()
    pltpu.sync_copy(acc.at[pl.ds(s * SL, SL)],
                    e2p.at[pl.ds(c * NPAD + s * SL, SL)])


def _seg_e2(rowp2, colp, e1p):
    mesh = plsc.VectorSubcoreMesh(core_axis_name="c", subcore_axis_name="s")
    f = pl.kernel(
        _seg_e2_body,
        out_type=jax.ShapeDtypeStruct((NC * NPAD,), jnp.float32),
        mesh=mesh,
        compiler_params=pltpu.CompilerParams(needs_layout_passes=False),
        scratch_types=[
            pltpu.VMEM_SHARED((NPAD,), jnp.float32),
            pltpu.VMEM((NPAD,), jnp.float32),
            pltpu.VMEM((NPAD,), jnp.float32),
            pltpu.VMEM((EW,), jnp.int32),
            pltpu.VMEM((NCHUNK, CHUNK), jnp.int32),
            pltpu.VMEM((EW,), jnp.float32),
            pltpu.VMEM((SL // 4,), jnp.float32),
            pltpu.SemaphoreType.DMA,
        ],
    )
    return f(rowp2, colp, e1p)


def _seg_aggr_body(rowp, colp, newx, aggrp,
                   acc, iv0, iv1, wv0, wv1, rv0, rv1,
                   zb, sg0, sg1, si0, si1):
    c, s, base = _worker_base()
    _zero_slice(zb, acc, s, SL)
    pltpu.sync_copy(colp.at[pl.ds(base, CHUNK)], iv0)
    pltpu.sync_copy(rowp.at[pl.ds(base, CHUNK)], wv0)
    off1 = pl.multiple_of(base + CHUNK, CHUNK)
    pltpu.sync_copy(colp.at[pl.ds(off1, CHUNK)], iv1)
    pltpu.sync_copy(rowp.at[pl.ds(off1, CHUNK)], wv1)
    pltpu.async_copy(newx.at[iv0], rv0, sg0)
    plsc.subcore_barrier()

    ivs, wvs, rvs = (iv0, iv1), (wv0, wv1), (rv0, rv1)
    sgs, sis = (sg0, sg1), (si0, si1)

    @pl.loop(0, NCHUNK // 2)
    def _(t):
        for b in range(2):
            k = t * 2 + b
            iv, wv, rv, sg, si = ivs[b], wvs[b], rvs[b], sgs[b], sis[b]
            ivn, wvn, rvn, sgn, sin = (ivs[b ^ 1], wvs[b ^ 1], rvs[b ^ 1],
                                       sgs[b ^ 1], sis[b ^ 1])
            if b == 1:
                @pl.when(t < NCHUNK // 2 - 1)
                def _():
                    pltpu.make_async_copy(colp.at[pl.ds(base, CHUNK)],
                                          ivn, sin).wait()
                    pltpu.make_async_copy(rowp.at[pl.ds(base, CHUNK)],
                                          wvn, sin).wait()
            else:
                @pl.when(t > 0)
                def _():
                    pltpu.make_async_copy(colp.at[pl.ds(base, CHUNK)],
                                          ivn, sin).wait()
                    pltpu.make_async_copy(rowp.at[pl.ds(base, CHUNK)],
                                          wvn, sin).wait()

            @pl.when(k + 1 < NCHUNK)
            def _():
                pltpu.async_copy(newx.at[ivn], rvn, sgn)

            pltpu.make_async_copy(newx.at[iv], rv, sg).wait()
            pltpu.sync_copy(rv, acc.at[wv], add=True)

            @pl.when(k + 2 < NCHUNK)
            def _():
                off = pl.multiple_of(base + (k + 2) * CHUNK, CHUNK)
                pltpu.async_copy(colp.at[pl.ds(off, CHUNK)], iv, si)
                pltpu.async_copy(rowp.at[pl.ds(off, CHUNK)], wv, si)

    plsc.subcore_barrier()
    pltpu.sync_copy(acc.at[pl.ds(s * SL, SL)],
                    aggrp.at[pl.ds(c * NPAD + s * SL, SL)])


def _seg_aggr(rowp, colp, newx):
    mesh = plsc.VectorSubcoreMesh(core_axis_name="c", subcore_axis_name="s")
    f = pl.kernel(
        _seg_aggr_body,
        out_type=jax.ShapeDtypeStruct((NC * NPAD, D), jnp.float32),
        mesh=mesh,
        scratch_types=[
            pltpu.VMEM_SHARED((NPAD, D), jnp.float32),
            pltpu.VMEM((CHUNK,), jnp.int32),
            pltpu.VMEM((CHUNK,), jnp.int32),
            pltpu.VMEM((CHUNK,), jnp.int32),
            pltpu.VMEM((CHUNK,), jnp.int32),
            pltpu.VMEM((CHUNK, D), jnp.float32),
            pltpu.VMEM((CHUNK, D), jnp.float32),
            pltpu.VMEM((16, D), jnp.float32),
            pltpu.SemaphoreType.DMA,
            pltpu.SemaphoreType.DMA,
            pltpu.SemaphoreType.DMA,
            pltpu.SemaphoreType.DMA,
        ],
    )
    return f(rowp, colp, newx)


def kernel(input, pre_edge_feat, adj, degree, W_fc, W0, b0, W1, b1, W2, b2):
    row = adj[0].astype(jnp.int32)
    col = adj[1].astype(jnp.int32)
    npad_e = EPAD - E
    pad_i = jnp.arange(npad_e, dtype=jnp.int32)
    row_p = jnp.concatenate([row, N + (pad_i % (NPAD - N))])
    col_p = jnp.concatenate([col, pad_i % N])

    deg2 = degree.reshape(N, 1)
    b0r = b0.reshape(1, HID)
    b1r = b1.reshape(HID, 1)
    b2r = b2.reshape(1, 1)

    x, aft, new_x, rdh, rdeg = _dense_pre(input, W_fc, W0, b0r, deg2)

    lt, rt = _edge_gather(row_p, col_p, aft)
    s = _edge_mlp(lt, rt, W1, b1r, W2, b2r)

    row_p2 = row_p.reshape(EPAD // CHUNK, CHUNK)
    e1p = _seg_e1(row_p2, s.reshape(EPAD))
    e2p = _seg_e2(row_p2, col_p, e1p)
    aggrp = _seg_aggr(row_p, col_p, new_x)

    aggr_x = _final_aggr(aggrp.reshape(NC * NPAD, D), x, pre_edge_feat, rdh)
    edge_out = _edge_out(e2p.reshape(NC, NPAD), rdeg)
    return (aggr_x, edge_out)


# speedup vs baseline: 1.2488x; 1.2488x over previous
import functools

import jax
import jax.numpy as jnp
from jax import lax
from jax.experimental import pallas as pl
from jax.experimental.pallas import tpu as pltpu
from jax.experimental.pallas import tpu_sc as plsc

N = 10000
E = 320000
D = 128
HID = 32
NC = 2
NS = 16
NW = NC * NS
NPAD = 10240
SL = NPAD // NS
EPAD = 327680
EW = EPAD // NW
CHUNK = 128
NCHUNK = EW // CHUNK

_HIGH = lax.Precision.HIGHEST


def _dense_pre_body(inp, wfc, w0, b0, deg, x_o, aft_o, newx_o, rdh_o, rdeg_o):
    a = inp[...]
    x = lax.dot_general(a, wfc[...], (((1,), (1,)), ((), ())), precision=_HIGH)
    af = lax.dot_general(a, w0[...], (((1,), (1,)), ((), ())), precision=_HIGH)
    af = af + b0[...]
    d = deg[...]
    dh = jnp.sqrt(d)
    x_o[...] = x
    aft_o[...] = af.T
    newx_o[...] = x / dh
    rdh_o[...] = 1.0 / dh
    rdeg_o[...] = 1.0 / d


def _dense_pre(inp, wfc, w0, b0, deg):
    blk = 1024
    grid = NPAD // blk
    return pl.pallas_call(
        _dense_pre_body,
        grid=(grid,),
        in_specs=[
            pl.BlockSpec((blk, D), lambda i: (i, 0)),
            pl.BlockSpec((D, D), lambda i: (0, 0)),
            pl.BlockSpec((HID, D), lambda i: (0, 0)),
            pl.BlockSpec((1, HID), lambda i: (0, 0)),
            pl.BlockSpec((blk, 1), lambda i: (i, 0)),
        ],
        out_specs=[
            pl.BlockSpec((blk, D), lambda i: (i, 0)),
            pl.BlockSpec((HID, blk), lambda i: (0, i)),
            pl.BlockSpec((blk, D), lambda i: (i, 0)),
            pl.BlockSpec((blk, 1), lambda i: (i, 0)),
            pl.BlockSpec((blk, 1), lambda i: (i, 0)),
        ],
        out_shape=[
            jax.ShapeDtypeStruct((N, D), jnp.float32),
            jax.ShapeDtypeStruct((HID, NPAD), jnp.float32),
            jax.ShapeDtypeStruct((N, D), jnp.float32),
            jax.ShapeDtypeStruct((N, 1), jnp.float32),
            jax.ShapeDtypeStruct((N, 1), jnp.float32),
        ],
    )(inp, wfc, w0, b0, deg)


def _edge_mlp_body(lt_r, rt_r, w1, b1, w2, b2, s_o):
    l = lt_r[...]
    r = rt_r[...]
    feat = jnp.concatenate([l, r, jnp.abs(l - r),
                            jnp.ones((1, l.shape[1]), jnp.float32)], axis=0)
    w1e = jnp.concatenate([w1[...], b1[...]], axis=1)
    h = lax.dot_general(w1e, feat, (((1,), (0,)), ((), ())), precision=_HIGH)
    h = jnp.maximum(h, 0.0)
    w2p = jnp.concatenate([w2[...], jnp.zeros((7, HID), jnp.float32)], axis=0)
    z = lax.dot_general(w2p, h, (((1,), (0,)), ((), ())), precision=_HIGH)
    z = z + b2[0, 0]
    s_o[...] = 1.0 / (1.0 + jnp.exp(-z[0:1, :]))


def _edge_mlp(lt, rt, w1, b1, w2, b2):
    blk = 8192
    grid = EPAD // blk
    return pl.pallas_call(
        _edge_mlp_body,
        grid=(grid,),
        in_specs=[
            pl.BlockSpec((HID, blk), lambda i: (0, i)),
            pl.BlockSpec((HID, blk), lambda i: (0, i)),
            pl.BlockSpec((HID, 3 * HID), lambda i: (0, 0)),
            pl.BlockSpec((HID, 1), lambda i: (0, 0)),
            pl.BlockSpec((1, HID), lambda i: (0, 0)),
            pl.BlockSpec((1, 1), lambda i: (0, 0)),
        ],
        out_specs=pl.BlockSpec((1, blk), lambda i: (0, i)),
        out_shape=jax.ShapeDtypeStruct((1, EPAD), jnp.float32),
    )(lt, rt, w1, b1, w2, b2)


def _final_aggr_body(p0, p1, x_r, pef, rdh, out):
    a = (p0[...] + p1[...]) * rdh[...]
    p = pef[...]
    out[...] = a * p + x_r[...] * (1.0 - p)


def _final_aggr(aggrp, x, pef, rdh):
    blk = 1024
    grid = NPAD // blk
    return pl.pallas_call(
        _final_aggr_body,
        grid=(grid,),
        in_specs=[
            pl.BlockSpec((blk, D), lambda i: (i, 0)),
            pl.BlockSpec((blk, D), lambda i: (i + NPAD // 1024, 0)),
            pl.BlockSpec((blk, D), lambda i: (i, 0)),
            pl.BlockSpec((blk, 1), lambda i: (i, 0)),
            pl.BlockSpec((blk, 1), lambda i: (i, 0)),
        ],
        out_specs=pl.BlockSpec((blk, D), lambda i: (i, 0)),
        out_shape=jax.ShapeDtypeStruct((N, D), jnp.float32),
    )(aggrp, aggrp, x, pef, rdh)


def _edge_out_body(e2p, rdeg, out):
    v = e2p[...]
    e2 = (v[0] + v[1])[None, :N]
    out[...] = rdeg[...] * e2


def _edge_out(e2p, rdeg):
    blk = 200
    grid = N // blk
    return pl.pallas_call(
        _edge_out_body,
        grid=(grid,),
        in_specs=[
            pl.BlockSpec((2, NPAD), lambda i: (0, 0)),
            pl.BlockSpec((blk, 1), lambda i: (i, 0)),
        ],
        out_specs=pl.BlockSpec((blk, N), lambda i: (i, 0)),
        out_shape=jax.ShapeDtypeStruct((N, N), jnp.float32),
    )(e2p, rdeg)


_Z16 = functools.partial(jnp.zeros, (16,), jnp.float32)


def _worker_base():
    c = lax.axis_index("c")
    s = lax.axis_index("s")
    wid = c * NS + s
    return c, s, wid * EW


def _zero_slice(zb, acc, s, width):
    n = zb.shape[0]
    if len(zb.shape) == 1:
        for j in range(n // 16):
            zb[pl.ds(j * 16, 16)] = _Z16()
        @pl.loop(0, width // n)
        def _(r):
            pltpu.sync_copy(zb, acc.at[pl.ds(s * width + r * n, n)])
    else:
        for i in range(n):
            for j in range(zb.shape[1] // 16):
                zb[i, pl.ds(j * 16, 16)] = _Z16()
        @pl.loop(0, width // n)
        def _(r):
            pltpu.sync_copy(zb, acc.at[pl.ds(s * width + r * n, n)])


def _edge_gather_body(rowp, colp, aft, lt_o, rt_o, af8,
                      i0, i1, ob0, ob1, si0, si1, so0, so1):
    c = lax.axis_index("c")
    s = lax.axis_index("s")
    wid = c * NS + s
    fg = wid % (HID // 8)
    er = wid // (HID // 8)
    erw = EPAD // 8
    echunk = 2048
    nch = erw // echunk
    frow = pl.multiple_of(fg * 8, 8)
    pltpu.sync_copy(aft.at[pl.ds(frow, 8), :], af8)

    idxb = (i0, i1)
    obs = (ob0, ob1)
    sis = (si0, si1)
    sos = (so0, so1)

    for idx_src, dst in ((rowp, lt_o), (colp, rt_o)):
        base = pl.multiple_of(er * erw, echunk)
        pltpu.sync_copy(idx_src.at[pl.ds(base, echunk)], i0)

        @pl.loop(0, nch // 2)
        def _(t):
            for b in range(2):
                k = t * 2 + b
                ib, ob_, si, so = idxb[b], obs[b], sis[b], sos[b]
                inx, sinx = idxb[b ^ 1], sis[b ^ 1]
                nxt = pl.multiple_of(base + (k + 1) * echunk, echunk)
                if b == 0:
                    pltpu.async_copy(idx_src.at[pl.ds(nxt, echunk)], inx, sinx)
                else:
                    @pl.when(t < nch // 2 - 1)
                    def _():
                        pltpu.async_copy(idx_src.at[pl.ds(nxt, echunk)],
                                         inx, sinx)
                if b == 1:
                    pltpu.make_async_copy(
                        idx_src.at[pl.ds(base, echunk)], ib, si).wait()
                else:
                    @pl.when(t > 0)
                    def _():
                        pltpu.make_async_copy(
                            idx_src.at[pl.ds(base, echunk)], ib, si).wait()
                @pl.when(k >= 2)
                def _():
                    pltpu.make_async_copy(
                        ob_, dst.at[pl.ds(frow, 8), pl.ds(base, echunk)],
                        so).wait()

                @pl.loop(0, echunk // 64)
                def _(g):
                    ivs = [ib[pl.ds(g * 64 + 16 * j, 16)] for j in range(4)]
                    for kk in range(8):
                        kv = jnp.full((16,), kk, jnp.int32)
                        vs = [plsc.load_gather(af8, [kv, iv]) for iv in ivs]
                        for j in range(4):
                            ob_[kk, pl.ds(g * 64 + 16 * j, 16)] = vs[j]

                off = pl.multiple_of(base + k * echunk, echunk)
                pltpu.async_copy(
                    ob_, dst.at[pl.ds(frow, 8), pl.ds(off, echunk)], so)

        pltpu.make_async_copy(
            ob0, dst.at[pl.ds(frow, 8), pl.ds(base, echunk)], so0).wait()
        pltpu.make_async_copy(
            ob1, dst.at[pl.ds(frow, 8), pl.ds(base, echunk)], so1).wait()


def _edge_gather(rowp, colp, aft):
    mesh = plsc.VectorSubcoreMesh(core_axis_name="c", subcore_axis_name="s")
    f = pl.kernel(
        _edge_gather_body,
        out_type=[
            jax.ShapeDtypeStruct((HID, EPAD), jnp.float32),
            jax.ShapeDtypeStruct((HID, EPAD), jnp.float32),
        ],
        mesh=mesh,
        compiler_params=pltpu.CompilerParams(needs_layout_passes=False),
        scratch_types=[
            pltpu.VMEM((8, NPAD), jnp.float32),
            pltpu.VMEM((2048,), jnp.int32),
            pltpu.VMEM((2048,), jnp.int32),
            pltpu.VMEM((8, 2048), jnp.float32),
            pltpu.VMEM((8, 2048), jnp.float32),
            pltpu.SemaphoreType.DMA,
            pltpu.SemaphoreType.DMA,
            pltpu.SemaphoreType.DMA,
            pltpu.SemaphoreType.DMA,
        ],
    )
    return f(rowp, colp, aft)


def _seg_e1_body(rowp2, sflat, e1p, acc, rowa, vala, zb, sem):
    c, s, base = _worker_base()
    cb = (c * NS + s) * NCHUNK
    pltpu.async_copy(rowp2.at[pl.ds(cb, NCHUNK), :], rowa, sem)
    pltpu.async_copy(sflat.at[pl.ds(base, EW)], vala, sem)
    _zero_slice(zb, acc, s, SL)
    pltpu.make_async_copy(rowp2.at[pl.ds(cb, NCHUNK), :], rowa, sem).wait()
    pltpu.make_async_copy(sflat.at[pl.ds(base, EW)], vala, sem).wait()
    plsc.subcore_barrier()

    @pl.loop(0, NCHUNK)
    def _(k):
        off = pl.multiple_of(k * CHUNK, CHUNK)
        pltpu.async_copy(vala.at[pl.ds(off, CHUNK)], acc.at[rowa.at[k]],
                         sem, add=True)

    pltpu.make_async_copy(sflat.at[pl.ds(base, EW)], vala, sem).wait()
    plsc.subcore_barrier()
    pltpu.sync_copy(acc.at[pl.ds(s * SL, SL)],
                    e1p.at[pl.ds(c * NPAD + s * SL, SL)])


def _seg_e1(rowp2, sflat):
    mesh = plsc.VectorSubcoreMesh(core_axis_name="c", subcore_axis_name="s")
    f = pl.kernel(
        _seg_e1_body,
        out_type=jax.ShapeDtypeStruct((NC * NPAD,), jnp.float32),
        mesh=mesh,
        scratch_types=[
            pltpu.VMEM_SHARED((NPAD,), jnp.float32),
            pltpu.VMEM((NCHUNK, CHUNK), jnp.int32),
            pltpu.VMEM((EW,), jnp.float32),
            pltpu.VMEM((SL // 4,), jnp.float32),
            pltpu.SemaphoreType.DMA,
        ],
    )
    return f(rowp2, sflat)


def _seg_e2_body(rowp2, colp, e1p, e2p, acc, e1a, e1b, cola, rowa, vala,
                 zb, sem):
    c, s, base = _worker_base()
    cb = (c * NS + s) * NCHUNK
    pltpu.async_copy(e1p.at[pl.ds(0, NPAD)], e1a, sem)
    pltpu.async_copy(e1p.at[pl.ds(NPAD, NPAD)], e1b, sem)
    pltpu.async_copy(colp.at[pl.ds(base, EW)], cola, sem)
    pltpu.async_copy(rowp2.at[pl.ds(cb, NCHUNK), :], rowa, sem)
    _zero_slice(zb, acc, s, SL)
    pltpu.make_async_copy(e1p.at[pl.ds(0, NPAD)], e1a, sem).wait()
    pltpu.make_async_copy(e1p.at[pl.ds(NPAD, NPAD)], e1b, sem).wait()
    pltpu.make_async_copy(colp.at[pl.ds(base, EW)], cola, sem).wait()
    pltpu.make_async_copy(rowp2.at[pl.ds(cb, NCHUNK), :], rowa, sem).wait()

    @pl.loop(0, EW // 16)
    def _(g):
        cv = cola[pl.ds(g * 16, 16)]
        vala[pl.ds(g * 16, 16)] = (plsc.load_gather(e1a, [cv]) +
                                   plsc.load_gather(e1b, [cv]))

    plsc.subcore_barrier()

    @pl.loop(0, NCHUNK)
    def _(k):
        off = pl.multiple_of(k * CHUNK, CHUNK)
        pltpu.async_copy(vala.at[pl.ds(off, CHUNK)], acc.at[rowa.at[k]],
                         sem, add=True)

    pltpu.make_async_copy(colp.at[pl.ds(base, EW)], vala, sem).wait()
    plsc.subcore_barrier()
    pltpu.sync_copy(acc.at[pl.ds(s * SL, SL)],
                    e2p.at[pl.ds(c * NPAD + s * SL, SL)])


def _seg_e2(rowp2, colp, e1p):
    mesh = plsc.VectorSubcoreMesh(core_axis_name="c", subcore_axis_name="s")
    f = pl.kernel(
        _seg_e2_body,
        out_type=jax.ShapeDtypeStruct((NC * NPAD,), jnp.float32),
        mesh=mesh,
        compiler_params=pltpu.CompilerParams(needs_layout_passes=False),
        scratch_types=[
            pltpu.VMEM_SHARED((NPAD,), jnp.float32),
            pltpu.VMEM((NPAD,), jnp.float32),
            pltpu.VMEM((NPAD,), jnp.float32),
            pltpu.VMEM((EW,), jnp.int32),
            pltpu.VMEM((NCHUNK, CHUNK), jnp.int32),
            pltpu.VMEM((EW,), jnp.float32),
            pltpu.VMEM((SL // 4,), jnp.float32),
            pltpu.SemaphoreType.DMA,
        ],
    )
    return f(rowp2, colp, e1p)


def _seg_aggr_body(rowp, colp, newx, aggrp,
                   acc, iv0, iv1, wv0, wv1, rv0, rv1,
                   zb, sg0, sg1, si0, si1):
    c, s, base = _worker_base()
    _zero_slice(zb, acc, s, SL)
    pltpu.sync_copy(colp.at[pl.ds(base, CHUNK)], iv0)
    pltpu.sync_copy(rowp.at[pl.ds(base, CHUNK)], wv0)
    off1 = pl.multiple_of(base + CHUNK, CHUNK)
    pltpu.sync_copy(colp.at[pl.ds(off1, CHUNK)], iv1)
    pltpu.sync_copy(rowp.at[pl.ds(off1, CHUNK)], wv1)
    pltpu.async_copy(newx.at[iv0], rv0, sg0)
    plsc.subcore_barrier()

    ivs, wvs, rvs = (iv0, iv1), (wv0, wv1), (rv0, rv1)
    sgs, sis = (sg0, sg1), (si0, si1)

    @pl.loop(0, NCHUNK // 2)
    def _(t):
        for b in range(2):
            k = t * 2 + b
            iv, wv, rv, sg, si = ivs[b], wvs[b], rvs[b], sgs[b], sis[b]
            ivn, wvn, rvn, sgn, sin = (ivs[b ^ 1], wvs[b ^ 1], rvs[b ^ 1],
                                       sgs[b ^ 1], sis[b ^ 1])
            if b == 1:
                @pl.when(t < NCHUNK // 2 - 1)
                def _():
                    pltpu.make_async_copy(colp.at[pl.ds(base, CHUNK)],
                                          ivn, sin).wait()
                    pltpu.make_async_copy(rowp.at[pl.ds(base, CHUNK)],
                                          wvn, sin).wait()
            else:
                @pl.when(t > 0)
                def _():
                    pltpu.make_async_copy(colp.at[pl.ds(base, CHUNK)],
                                          ivn, sin).wait()
                    pltpu.make_async_copy(rowp.at[pl.ds(base, CHUNK)],
                                          wvn, sin).wait()

            @pl.when(k + 1 < NCHUNK)
            def _():
                pltpu.async_copy(newx.at[ivn], rvn, sgn)

            pltpu.make_async_copy(newx.at[iv], rv, sg).wait()
            pltpu.sync_copy(rv, acc.at[wv], add=True)

            @pl.when(k + 2 < NCHUNK)
            def _():
                off = pl.multiple_of(base + (k + 2) * CHUNK, CHUNK)
                pltpu.async_copy(colp.at[pl.ds(off, CHUNK)], iv, si)
                pltpu.async_copy(rowp.at[pl.ds(off, CHUNK)], wv, si)

    plsc.subcore_barrier()
    pltpu.sync_copy(acc.at[pl.ds(s * SL, SL)],
                    aggrp.at[pl.ds(c * NPAD + s * SL, SL)])


def _seg_aggr(rowp, colp, newx):
    mesh = plsc.VectorSubcoreMesh(core_axis_name="c", subcore_axis_name="s")
    f = pl.kernel(
        _seg_aggr_body,
        out_type=jax.ShapeDtypeStruct((NC * NPAD, D), jnp.float32),
        mesh=mesh,
        scratch_types=[
            pltpu.VMEM_SHARED((NPAD, D), jnp.float32),
            pltpu.VMEM((CHUNK,), jnp.int32),
            pltpu.VMEM((CHUNK,), jnp.int32),
            pltpu.VMEM((CHUNK,), jnp.int32),
            pltpu.VMEM((CHUNK,), jnp.int32),
            pltpu.VMEM((CHUNK, D), jnp.float32),
            pltpu.VMEM((CHUNK, D), jnp.float32),
            pltpu.VMEM((16, D), jnp.float32),
            pltpu.SemaphoreType.DMA,
            pltpu.SemaphoreType.DMA,
            pltpu.SemaphoreType.DMA,
            pltpu.SemaphoreType.DMA,
        ],
    )
    return f(rowp, colp, newx)


def kernel(input, pre_edge_feat, adj, degree, W_fc, W0, b0, W1, b1, W2, b2):
    row = adj[0].astype(jnp.int32)
    col = adj[1].astype(jnp.int32)
    npad_e = EPAD - E
    pad_i = jnp.arange(npad_e, dtype=jnp.int32)
    row_p = jnp.concatenate([row, N + (pad_i % (NPAD - N))])
    col_p = jnp.concatenate([col, pad_i % N])

    deg2 = degree.reshape(N, 1)
    b0r = b0.reshape(1, HID)
    b1r = b1.reshape(HID, 1)
    b2r = b2.reshape(1, 1)

    x, aft, new_x, rdh, rdeg = _dense_pre(input, W_fc, W0, b0r, deg2)

    lt, rt = _edge_gather(row_p, col_p, aft)
    s = _edge_mlp(lt, rt, W1, b1r, W2, b2r)

    row_p2 = row_p.reshape(EPAD // CHUNK, CHUNK)
    e1p = _seg_e1(row_p2, s.reshape(EPAD))
    e2p = _seg_e2(row_p2, col_p, e1p)
    aggrp = _seg_aggr(row_p, col_p, new_x)

    aggr_x = _final_aggr(aggrp.reshape(NC * NPAD, D), x, pre_edge_feat, rdh)
    edge_out = _edge_out(e2p.reshape(NC, NPAD), rdeg)
    return (aggr_x, edge_out)


# speedup vs baseline: 1.2979x; 1.0393x over previous
import functools

import jax
import jax.numpy as jnp
from jax import lax
from jax.experimental import pallas as pl
from jax.experimental.pallas import tpu as pltpu
from jax.experimental.pallas import tpu_sc as plsc

N = 10000
E = 320000
D = 128
HID = 32
NC = 2
NS = 16
NW = NC * NS
NPAD = 10240
SL = NPAD // NS
EPAD = 327680
EW = EPAD // NW
CHUNK = 128
NCHUNK = EW // CHUNK

_HIGH = lax.Precision.HIGHEST


def _dense_pre_body(inp, wfc, w0, b0, deg, x_o, aft_o, newx_o, rdh_o, rdeg_o):
    a = inp[...]
    x = lax.dot_general(a, wfc[...], (((1,), (1,)), ((), ())), precision=_HIGH)
    af = lax.dot_general(a, w0[...], (((1,), (1,)), ((), ())), precision=_HIGH)
    af = af + b0[...]
    d = deg[...]
    dh = jnp.sqrt(d)
    x_o[...] = x
    aft_o[...] = af.T
    newx_o[...] = x / dh
    rdh_o[...] = 1.0 / dh
    rdeg_o[...] = 1.0 / d


def _dense_pre(inp, wfc, w0, b0, deg):
    blk = 1024
    grid = NPAD // blk
    return pl.pallas_call(
        _dense_pre_body,
        grid=(grid,),
        in_specs=[
            pl.BlockSpec((blk, D), lambda i: (i, 0)),
            pl.BlockSpec((D, D), lambda i: (0, 0)),
            pl.BlockSpec((HID, D), lambda i: (0, 0)),
            pl.BlockSpec((1, HID), lambda i: (0, 0)),
            pl.BlockSpec((blk, 1), lambda i: (i, 0)),
        ],
        out_specs=[
            pl.BlockSpec((blk, D), lambda i: (i, 0)),
            pl.BlockSpec((HID, blk), lambda i: (0, i)),
            pl.BlockSpec((blk, D), lambda i: (i, 0)),
            pl.BlockSpec((blk, 1), lambda i: (i, 0)),
            pl.BlockSpec((blk, 1), lambda i: (i, 0)),
        ],
        out_shape=[
            jax.ShapeDtypeStruct((N, D), jnp.float32),
            jax.ShapeDtypeStruct((HID, NPAD), jnp.float32),
            jax.ShapeDtypeStruct((N, D), jnp.float32),
            jax.ShapeDtypeStruct((N, 1), jnp.float32),
            jax.ShapeDtypeStruct((N, 1), jnp.float32),
        ],
    )(inp, wfc, w0, b0, deg)


def _edge_mlp_body(lt_r, rt_r, w1, b1, w2, b2, s_o):
    l = lt_r[...]
    r = rt_r[...]
    feat = jnp.concatenate([l, r, jnp.abs(l - r),
                            jnp.ones((1, l.shape[1]), jnp.float32)], axis=0)
    w1e = jnp.concatenate([w1[...], b1[...]], axis=1)
    h = lax.dot_general(w1e, feat, (((1,), (0,)), ((), ())), precision=_HIGH)
    h = jnp.maximum(h, 0.0)
    w2p = jnp.concatenate([w2[...], jnp.zeros((7, HID), jnp.float32)], axis=0)
    z = lax.dot_general(w2p, h, (((1,), (0,)), ((), ())), precision=_HIGH)
    z = z + b2[0, 0]
    s_o[...] = 1.0 / (1.0 + jnp.exp(-z[0:1, :]))


def _edge_mlp(lt, rt, w1, b1, w2, b2):
    blk = 8192
    grid = EPAD // blk
    return pl.pallas_call(
        _edge_mlp_body,
        grid=(grid,),
        in_specs=[
            pl.BlockSpec((HID, blk), lambda i: (0, i)),
            pl.BlockSpec((HID, blk), lambda i: (0, i)),
            pl.BlockSpec((HID, 3 * HID), lambda i: (0, 0)),
            pl.BlockSpec((HID, 1), lambda i: (0, 0)),
            pl.BlockSpec((1, HID), lambda i: (0, 0)),
            pl.BlockSpec((1, 1), lambda i: (0, 0)),
        ],
        out_specs=pl.BlockSpec((1, blk), lambda i: (0, i)),
        out_shape=jax.ShapeDtypeStruct((1, EPAD), jnp.float32),
    )(lt, rt, w1, b1, w2, b2)


def _final_aggr_body(p0, p1, x_r, pef, rdh, out):
    a = (p0[...] + p1[...]) * rdh[...]
    p = pef[...]
    out[...] = a * p + x_r[...] * (1.0 - p)


def _final_aggr(aggrp, x, pef, rdh):
    blk = 1024
    grid = NPAD // blk
    return pl.pallas_call(
        _final_aggr_body,
        grid=(grid,),
        in_specs=[
            pl.BlockSpec((blk, D), lambda i: (i, 0)),
            pl.BlockSpec((blk, D), lambda i: (i + NPAD // 1024, 0)),
            pl.BlockSpec((blk, D), lambda i: (i, 0)),
            pl.BlockSpec((blk, 1), lambda i: (i, 0)),
            pl.BlockSpec((blk, 1), lambda i: (i, 0)),
        ],
        out_specs=pl.BlockSpec((blk, D), lambda i: (i, 0)),
        out_shape=jax.ShapeDtypeStruct((N, D), jnp.float32),
    )(aggrp, aggrp, x, pef, rdh)


def _edge_out_body(e2p, rdeg, out):
    v = e2p[...]
    e2 = (v[0] + v[1])[None, :N]
    out[...] = rdeg[...] * e2


def _edge_out(e2p, rdeg):
    blk = 200
    grid = N // blk
    return pl.pallas_call(
        _edge_out_body,
        grid=(grid,),
        in_specs=[
            pl.BlockSpec((2, NPAD), lambda i: (0, 0)),
            pl.BlockSpec((blk, 1), lambda i: (i, 0)),
        ],
        out_specs=pl.BlockSpec((blk, N), lambda i: (i, 0)),
        out_shape=jax.ShapeDtypeStruct((N, N), jnp.float32),
    )(e2p, rdeg)


_Z16 = functools.partial(jnp.zeros, (16,), jnp.float32)


def _worker_base():
    c = lax.axis_index("c")
    s = lax.axis_index("s")
    wid = c * NS + s
    return c, s, wid * EW


def _zero_slice(zb, acc, s, width):
    n = zb.shape[0]
    if len(zb.shape) == 1:
        for j in range(n // 16):
            zb[pl.ds(j * 16, 16)] = _Z16()
        @pl.loop(0, width // n)
        def _(r):
            pltpu.sync_copy(zb, acc.at[pl.ds(s * width + r * n, n)])
    else:
        for i in range(n):
            for j in range(zb.shape[1] // 16):
                zb[i, pl.ds(j * 16, 16)] = _Z16()
        @pl.loop(0, width // n)
        def _(r):
            pltpu.sync_copy(zb, acc.at[pl.ds(s * width + r * n, n)])


def _edge_gather_body(rowp, colp, aft, lt_o, rt_o, af8,
                      i0, i1, ob0, ob1, si0, si1, so0, so1):
    c = lax.axis_index("c")
    s = lax.axis_index("s")
    wid = c * NS + s
    fg = wid % (HID // 8)
    er = wid // (HID // 8)
    erw = EPAD // 8
    echunk = 2048
    nch = erw // echunk
    frow = pl.multiple_of(fg * 8, 8)
    pltpu.sync_copy(aft.at[pl.ds(frow, 8), :], af8)

    idxb = (i0, i1)
    obs = (ob0, ob1)
    sis = (si0, si1)
    sos = (so0, so1)

    for idx_src, dst in ((rowp, lt_o), (colp, rt_o)):
        base = pl.multiple_of(er * erw, echunk)
        pltpu.sync_copy(idx_src.at[pl.ds(base, echunk)], i0)

        @pl.loop(0, nch // 2)
        def _(t):
            for b in range(2):
                k = t * 2 + b
                ib, ob_, si, so = idxb[b], obs[b], sis[b], sos[b]
                inx, sinx = idxb[b ^ 1], sis[b ^ 1]
                nxt = pl.multiple_of(base + (k + 1) * echunk, echunk)
                if b == 0:
                    pltpu.async_copy(idx_src.at[pl.ds(nxt, echunk)], inx, sinx)
                else:
                    @pl.when(t < nch // 2 - 1)
                    def _():
                        pltpu.async_copy(idx_src.at[pl.ds(nxt, echunk)],
                                         inx, sinx)
                if b == 1:
                    pltpu.make_async_copy(
                        idx_src.at[pl.ds(base, echunk)], ib, si).wait()
                else:
                    @pl.when(t > 0)
                    def _():
                        pltpu.make_async_copy(
                            idx_src.at[pl.ds(base, echunk)], ib, si).wait()
                @pl.when(k >= 2)
                def _():
                    pltpu.make_async_copy(
                        ob_, dst.at[pl.ds(frow, 8), pl.ds(base, echunk)],
                        so).wait()

                @pl.loop(0, echunk // 128)
                def _(g):
                    ivs = [ib[pl.ds(g * 128 + 16 * j, 16)] for j in range(8)]
                    for kk in range(8):
                        kv = jnp.full((16,), kk, jnp.int32)
                        vs = [plsc.load_gather(af8, [kv, iv]) for iv in ivs]
                        for j in range(8):
                            ob_[kk, pl.ds(g * 128 + 16 * j, 16)] = vs[j]

                off = pl.multiple_of(base + k * echunk, echunk)
                pltpu.async_copy(
                    ob_, dst.at[pl.ds(frow, 8), pl.ds(off, echunk)], so)

        pltpu.make_async_copy(
            ob0, dst.at[pl.ds(frow, 8), pl.ds(base, echunk)], so0).wait()
        pltpu.make_async_copy(
            ob1, dst.at[pl.ds(frow, 8), pl.ds(base, echunk)], so1).wait()


def _edge_gather(rowp, colp, aft):
    mesh = plsc.VectorSubcoreMesh(core_axis_name="c", subcore_axis_name="s")
    f = pl.kernel(
        _edge_gather_body,
        out_type=[
            jax.ShapeDtypeStruct((HID, EPAD), jnp.float32),
            jax.ShapeDtypeStruct((HID, EPAD), jnp.float32),
        ],
        mesh=mesh,
        compiler_params=pltpu.CompilerParams(needs_layout_passes=False),
        scratch_types=[
            pltpu.VMEM((8, NPAD), jnp.float32),
            pltpu.VMEM((2048,), jnp.int32),
            pltpu.VMEM((2048,), jnp.int32),
            pltpu.VMEM((8, 2048), jnp.float32),
            pltpu.VMEM((8, 2048), jnp.float32),
            pltpu.SemaphoreType.DMA,
            pltpu.SemaphoreType.DMA,
            pltpu.SemaphoreType.DMA,
            pltpu.SemaphoreType.DMA,
        ],
    )
    return f(rowp, colp, aft)


def _seg_e1_body(rowp2, sflat, e1p, acc, rowa, vala, zb, sem):
    c, s, base = _worker_base()
    cb = (c * NS + s) * NCHUNK
    pltpu.async_copy(rowp2.at[pl.ds(cb, NCHUNK), :], rowa, sem)
    pltpu.async_copy(sflat.at[pl.ds(base, EW)], vala, sem)
    _zero_slice(zb, acc, s, SL)
    pltpu.make_async_copy(rowp2.at[pl.ds(cb, NCHUNK), :], rowa, sem).wait()
    pltpu.make_async_copy(sflat.at[pl.ds(base, EW)], vala, sem).wait()
    plsc.subcore_barrier()

    @pl.loop(0, NCHUNK)
    def _(k):
        off = pl.multiple_of(k * CHUNK, CHUNK)
        pltpu.async_copy(vala.at[pl.ds(off, CHUNK)], acc.at[rowa.at[k]],
                         sem, add=True)

    pltpu.make_async_copy(sflat.at[pl.ds(base, EW)], vala, sem).wait()
    plsc.subcore_barrier()
    pltpu.sync_copy(acc.at[pl.ds(s * SL, SL)],
                    e1p.at[pl.ds(c * NPAD + s * SL, SL)])


def _seg_e1(rowp2, sflat):
    mesh = plsc.VectorSubcoreMesh(core_axis_name="c", subcore_axis_name="s")
    f = pl.kernel(
        _seg_e1_body,
        out_type=jax.ShapeDtypeStruct((NC * NPAD,), jnp.float32),
        mesh=mesh,
        scratch_types=[
            pltpu.VMEM_SHARED((NPAD,), jnp.float32),
            pltpu.VMEM((NCHUNK, CHUNK), jnp.int32),
            pltpu.VMEM((EW,), jnp.float32),
            pltpu.VMEM((SL // 4,), jnp.float32),
            pltpu.SemaphoreType.DMA,
        ],
    )
    return f(rowp2, sflat)


def _seg_e2_body(rowp2, colp, e1p, e2p, acc, e1a, e1b, cola, rowa, vala,
                 zb, sem):
    c, s, base = _worker_base()
    cb = (c * NS + s) * NCHUNK
    pltpu.async_copy(e1p.at[pl.ds(0, NPAD)], e1a, sem)
    pltpu.async_copy(e1p.at[pl.ds(NPAD, NPAD)], e1b, sem)
    pltpu.async_copy(colp.at[pl.ds(base, EW)], cola, sem)
    pltpu.async_copy(rowp2.at[pl.ds(cb, NCHUNK), :], rowa, sem)
    _zero_slice(zb, acc, s, SL)
    pltpu.make_async_copy(e1p.at[pl.ds(0, NPAD)], e1a, sem).wait()
    pltpu.make_async_copy(e1p.at[pl.ds(NPAD, NPAD)], e1b, sem).wait()
    pltpu.make_async_copy(colp.at[pl.ds(base, EW)], cola, sem).wait()
    pltpu.make_async_copy(rowp2.at[pl.ds(cb, NCHUNK), :], rowa, sem).wait()

    @pl.loop(0, EW // 16)
    def _(g):
        cv = cola[pl.ds(g * 16, 16)]
        vala[pl.ds(g * 16, 16)] = (plsc.load_gather(e1a, [cv]) +
                                   plsc.load_gather(e1b, [cv]))

    plsc.subcore_barrier()

    @pl.loop(0, NCHUNK)
    def _(k):
        off = pl.multiple_of(k * CHUNK, CHUNK)
        pltpu.async_copy(vala.at[pl.ds(off, CHUNK)], acc.at[rowa.at[k]],
                         sem, add=True)

    pltpu.make_async_copy(colp.at[pl.ds(base, EW)], vala, sem).wait()
    plsc.subcore_barrier()
    pltpu.sync_copy(acc.at[pl.ds(s * SL, SL)],
                    e2p.at[pl.ds(c * NPAD + s * SL, SL)])


def _seg_e2(rowp2, colp, e1p):
    mesh = plsc.VectorSubcoreMesh(core_axis_name="c", subcore_axis_name="s")
    f = pl.kernel(
        _seg_e2_body,
        out_type=jax.ShapeDtypeStruct((NC * NPAD,), jnp.float32),
        mesh=mesh,
        compiler_params=pltpu.CompilerParams(needs_layout_passes=False),
        scratch_types=[
            pltpu.VMEM_SHARED((NPAD,), jnp.float32),
            pltpu.VMEM((NPAD,), jnp.float32),
            pltpu.VMEM((NPAD,), jnp.float32),
            pltpu.VMEM((EW,), jnp.int32),
            pltpu.VMEM((NCHUNK, CHUNK), jnp.int32),
            pltpu.VMEM((EW,), jnp.float32),
            pltpu.VMEM((SL // 4,), jnp.float32),
            pltpu.SemaphoreType.DMA,
        ],
    )
    return f(rowp2, colp, e1p)


def _seg_aggr_body(rowp, colp, newx, aggrp,
                   acc, iv0, iv1, wv0, wv1, rv0, rv1,
                   zb, sg0, sg1, si0, si1):
    c, s, base = _worker_base()
    _zero_slice(zb, acc, s, SL)
    pltpu.sync_copy(colp.at[pl.ds(base, CHUNK)], iv0)
    pltpu.sync_copy(rowp.at[pl.ds(base, CHUNK)], wv0)
    off1 = pl.multiple_of(base + CHUNK, CHUNK)
    pltpu.sync_copy(colp.at[pl.ds(off1, CHUNK)], iv1)
    pltpu.sync_copy(rowp.at[pl.ds(off1, CHUNK)], wv1)
    pltpu.async_copy(newx.at[iv0], rv0, sg0)
    plsc.subcore_barrier()

    ivs, wvs, rvs = (iv0, iv1), (wv0, wv1), (rv0, rv1)
    sgs, sis = (sg0, sg1), (si0, si1)

    @pl.loop(0, NCHUNK // 2)
    def _(t):
        for b in range(2):
            k = t * 2 + b
            iv, wv, rv, sg, si = ivs[b], wvs[b], rvs[b], sgs[b], sis[b]
            ivn, wvn, rvn, sgn, sin = (ivs[b ^ 1], wvs[b ^ 1], rvs[b ^ 1],
                                       sgs[b ^ 1], sis[b ^ 1])
            if b == 1:
                @pl.when(t < NCHUNK // 2 - 1)
                def _():
                    pltpu.make_async_copy(colp.at[pl.ds(base, CHUNK)],
                                          ivn, sin).wait()
                    pltpu.make_async_copy(rowp.at[pl.ds(base, CHUNK)],
                                          wvn, sin).wait()
            else:
                @pl.when(t > 0)
                def _():
                    pltpu.make_async_copy(colp.at[pl.ds(base, CHUNK)],
                                          ivn, sin).wait()
                    pltpu.make_async_copy(rowp.at[pl.ds(base, CHUNK)],
                                          wvn, sin).wait()

            @pl.when(k + 1 < NCHUNK)
            def _():
                pltpu.async_copy(newx.at[ivn], rvn, sgn)

            pltpu.make_async_copy(newx.at[iv], rv, sg).wait()
            pltpu.sync_copy(rv, acc.at[wv], add=True)

            @pl.when(k + 2 < NCHUNK)
            def _():
                off = pl.multiple_of(base + (k + 2) * CHUNK, CHUNK)
                pltpu.async_copy(colp.at[pl.ds(off, CHUNK)], iv, si)
                pltpu.async_copy(rowp.at[pl.ds(off, CHUNK)], wv, si)

    plsc.subcore_barrier()
    pltpu.sync_copy(acc.at[pl.ds(s * SL, SL)],
                    aggrp.at[pl.ds(c * NPAD + s * SL, SL)])


def _seg_aggr(rowp, colp, newx):
    mesh = plsc.VectorSubcoreMesh(core_axis_name="c", subcore_axis_name="s")
    f = pl.kernel(
        _seg_aggr_body,
        out_type=jax.ShapeDtypeStruct((NC * NPAD, D), jnp.float32),
        mesh=mesh,
        scratch_types=[
            pltpu.VMEM_SHARED((NPAD, D), jnp.float32),
            pltpu.VMEM((CHUNK,), jnp.int32),
            pltpu.VMEM((CHUNK,), jnp.int32),
            pltpu.VMEM((CHUNK,), jnp.int32),
            pltpu.VMEM((CHUNK,), jnp.int32),
            pltpu.VMEM((CHUNK, D), jnp.float32),
            pltpu.VMEM((CHUNK, D), jnp.float32),
            pltpu.VMEM((16, D), jnp.float32),
            pltpu.SemaphoreType.DMA,
            pltpu.SemaphoreType.DMA,
            pltpu.SemaphoreType.DMA,
            pltpu.SemaphoreType.DMA,
        ],
    )
    return f(rowp, colp, newx)


def kernel(input, pre_edge_feat, adj, degree, W_fc, W0, b0, W1, b1, W2, b2):
    row = adj[0].astype(jnp.int32)
    col = adj[1].astype(jnp.int32)
    npad_e = EPAD - E
    pad_i = jnp.arange(npad_e, dtype=jnp.int32)
    row_p = jnp.concatenate([row, N + (pad_i % (NPAD - N))])
    col_p = jnp.concatenate([col, pad_i % N])

    deg2 = degree.reshape(N, 1)
    b0r = b0.reshape(1, HID)
    b1r = b1.reshape(HID, 1)
    b2r = b2.reshape(1, 1)

    x, aft, new_x, rdh, rdeg = _dense_pre(input, W_fc, W0, b0r, deg2)

    lt, rt = _edge_gather(row_p, col_p, aft)
    s = _edge_mlp(lt, rt, W1, b1r, W2, b2r)

    row_p2 = row_p.reshape(EPAD // CHUNK, CHUNK)
    e1p = _seg_e1(row_p2, s.reshape(EPAD))
    e2p = _seg_e2(row_p2, col_p, e1p)
    aggrp = _seg_aggr(row_p, col_p, new_x)

    aggr_x = _final_aggr(aggrp.reshape(NC * NPAD, D), x, pre_edge_feat, rdh)
    edge_out = _edge_out(e2p.reshape(NC, NPAD), rdeg)
    return (aggr_x, edge_out)
